# baseline (speedup 1.0000x reference)
import functools
import math

import jax
import jax.numpy as jnp
import numpy as np
from jax import lax
from jax.experimental import pallas as pl
from jax.experimental.pallas import tpu as pltpu
from jax.experimental.pallas import tpu_sc as plsc

MAXH = 3
G = 512
NCLS = 4
DUMP_A = NCLS * G
ACC_A_ROWS = 129 * 16
GB = 520
DUMP_B = 512
NC, NS = 2, 16
NW = NC * NS
CH = 128

_PI2 = np.float32(math.pi / 2.0)
W_HOP = tuple(
    float(np.float32(math.cos(np.float32(_PI2 * np.float32(h) / np.float32(3.0)))))
    for h in range(NCLS)
)


def _hclass(hp):
    return jnp.where((hp >= 1) & (hp <= MAXH), hp, 0)


def _atom_phase(w, feats, hop, seg, buf, segb, hopb, idxb, acc, n, S):
    lo = (((w * n) // NW) // 8) * 8
    hi = ((((w + 1) * n) // NW) // 8) * 8
    nfull = (hi - lo) // S
    max_rng = n // NW + 8
    nst = -(-max_rng // S)

    def stage(t, carry):
        full = t < nfull
        base = pl.multiple_of(jnp.where(full, lo + t * S, hi - S), 8)
        done = jnp.where(full, base, lo + nfull * S)
        pltpu.sync_copy(feats.at[pl.ds(base, S)], buf)
        pltpu.sync_copy(seg.at[pl.ds(base, S)], segb.at[pl.ds(0, S)])
        pltpu.sync_copy(hop.at[pl.ds(base, S)], hopb.at[pl.ds(0, S)])

        def idxk(k, c2):
            sg = segb[pl.ds(k * 16, 16)]
            hp = hopb[pl.ds(k * 16, 16)]
            row = _hclass(hp) * G + sg
            gidx = base + k * 16 + lax.iota(jnp.int32, 16)
            row = jnp.where(gidx < done, DUMP_A, row)
            idxb[k // 8, pl.ds((k % 8) * 16, 16)] = row
            return c2

        lax.fori_loop(0, S // 16, idxk, 0)
        for j in range(S // CH):
            pltpu.sync_copy(buf.at[pl.ds(j * CH, CH)], acc.at[idxb.at[j]], add=True)
        return carry

    lax.fori_loop(0, nst, stage, 0)


def _bond_phase(w, feats4, hop, seg, bbuf, segb, hopb, accl, tscr, nblk, SB):
    lo = (w * nblk) // NW
    hi = ((w + 1) * nblk) // NW
    nfull = (hi - lo) // SB
    max_rng = nblk // NW + 1
    nst = -(-max_rng // SB)

    f16 = lax.iota(jnp.int32, 16)
    ca = f16 // 8
    cc = f16 % 8
    crow = ca * (SB * 8) + cc
    i16 = lax.iota(jnp.int32, 16)
    half = nblk * 8

    def stage(t, carry):
        full = t < nfull
        base = jnp.where(full, lo + t * SB, hi - SB)
        done = jnp.where(full, base, lo + nfull * SB) * 128
        baseb = base * 128
        for a in range(2):
            pltpu.sync_copy(feats4.at[pl.ds(a * half + base * 8, SB * 8)],
                            bbuf.at[pl.ds(a * SB * 8, SB * 8)])
        pltpu.sync_copy(seg.at[pl.ds(baseb, SB * 128)], segb)
        pltpu.sync_copy(hop.at[pl.ds(baseb, SB * 128)], hopb)

        def tgtk(k, c2):
            j16 = k * 16
            sg = segb[pl.ds(j16, 16)]
            hp = hopb[pl.ds(j16, 16)]
            tgt = _hclass(hp) * GB + sg
            gidx = baseb + j16 + i16
            tscr[pl.ds(j16, 16)] = jnp.where(gidx < done, DUMP_B, tgt)
            return c2

        lax.fori_loop(0, SB * 8, tgtk, 0)

        def group(g, c2):
            j16 = g * 16
            jv = jnp.full((16,), j16, jnp.int32)
            rowv = crow + (g // 8) * 8
            dbase = jnp.full((16,), j16 % 128, jnp.int32)
            for u in range(16):
                col = plsc.load_gather(bbuf, [rowv, dbase + u])
                tv = plsc.load_gather(tscr, [jv + u])
                plsc.addupdate_scatter(accl, [i16, tv], col)
            return c2

        lax.fori_loop(0, SB * 8, group, 0)
        return carry

    lax.fori_loop(0, nst, stage, 0)


@functools.lru_cache(maxsize=None)
def _make_sc(natom, nbond, da, db):
    sa = 256
    sbb = 16
    nblk = nbond // 128
    assert natom % 8 == 0 and nbond % 128 == 0
    assert natom // NW - 8 >= sa and nblk // NW >= sbb
    assert da == 128 and db == 16

    mesh = plsc.VectorSubcoreMesh(core_axis_name="c", subcore_axis_name="s")

    @functools.partial(
        pl.kernel,
        out_type=(
            jax.ShapeDtypeStruct((NC, NCLS * G, da), jnp.float32),
            jax.ShapeDtypeStruct((NC, 16, NCLS * GB), jnp.float32),
        ),
        mesh=mesh,
        compiler_params=pltpu.CompilerParams(
            use_tc_tiling_on_sc=False, needs_layout_passes=False),
        scratch_types=[
            pltpu.VMEM((sa, da), jnp.float32),
            pltpu.VMEM((2 * sbb * 8, 128), jnp.float32),
            pltpu.VMEM((sbb * 128,), jnp.int32),
            pltpu.VMEM((sbb * 128,), jnp.int32),
            pltpu.VMEM((sa // CH, CH), jnp.int32),
            pltpu.VMEM((16, NCLS * GB), jnp.float32),
            pltpu.VMEM((16,), jnp.int32),
            pltpu.VMEM((sbb * 128,), jnp.int32),
            pltpu.VMEM_SHARED((ACC_A_ROWS, da), jnp.float32),
            pltpu.VMEM_SHARED((16, NCLS * GB), jnp.float32),
        ],
    )
    def sc(af, ah, asg, bf4, bh, bsg, za, zbt, pa, pb,
           abuf, bbuf, segb, hopb, idxb, accl, rix, tscr, acc_a, acc_bt):
        c = lax.axis_index("c")
        s = lax.axis_index("s")
        w = c * NS + s
        pltpu.sync_copy(za, acc_a.at[pl.ds(s * 129, 129)])
        pltpu.sync_copy(zbt.at[pl.ds(s, 1)], acc_bt.at[pl.ds(s, 1)])

        def zl(k, c2):
            accl[k // (NCLS * GB // 16),
                 pl.ds((k % (NCLS * GB // 16)) * 16, 16)] = jnp.zeros(
                     (16,), jnp.float32)
            return c2

        lax.fori_loop(0, NCLS * GB, zl, 0)
        rix[...] = lax.iota(jnp.int32, 16)
        plsc.subcore_barrier()

        _atom_phase(w, af, ah, asg, abuf, segb, hopb, idxb, acc_a, natom, sa)
        _bond_phase(w, bf4, bh, bsg, bbuf, segb, hopb, accl, tscr, nblk, sbb)
        pltpu.sync_copy(accl, acc_bt.at[rix], add=True)
        plsc.subcore_barrier()
        pltpu.sync_copy(acc_a.at[pl.ds(s * 128, 128)], abuf.at[pl.ds(0, 128)])
        pltpu.sync_copy(abuf.at[pl.ds(0, 128)], pa.at[c, pl.ds(s * 128, 128)])
        pltpu.sync_copy(acc_bt.at[pl.ds(s, 1)], pb.at[c, pl.ds(s, 1)])

    return sc


def _combine_body(pa_ref, pbt_ref, oa_ref, obt_ref):
    da = pa_ref.shape[-1]
    oa = jnp.zeros((G, da), jnp.float32)
    obt = jnp.zeros((16, G), jnp.float32)
    for c in range(NC):
        for h in range(NCLS):
            oa = oa + W_HOP[h] * pa_ref[c, h * G:(h + 1) * G, :]
            obt = obt + W_HOP[h] * pbt_ref[c, :, h * GB:h * GB + G]
    oa_ref[...] = oa
    obt_ref[...] = obt


def kernel(atom_feats, bond_feats, global_feats, atom_hop_distance,
           bond_hop_distance, atom_segment_ids, bond_segment_ids):
    natom, da = atom_feats.shape
    nbond, db = bond_feats.shape
    nblk = nbond // 128
    bf4 = jnp.reshape(
        jnp.transpose(
            jnp.reshape(jnp.transpose(bond_feats), (2, 8, nblk, 128)),
            (0, 2, 1, 3)),
        (2 * nblk * 8, 128))
    za = jnp.zeros((129, da), jnp.float32)
    zbt = jnp.zeros((16, NCLS * GB), jnp.float32)
    sc = _make_sc(natom, nbond, da, db)
    pa, pb = sc(atom_feats, atom_hop_distance.astype(jnp.int32),
                atom_segment_ids.astype(jnp.int32), bf4,
                bond_hop_distance.astype(jnp.int32),
                bond_segment_ids.astype(jnp.int32), za, zbt)
    oa, obt = pl.pallas_call(
        _combine_body,
        out_shape=[
            jax.ShapeDtypeStruct((G, da), jnp.float32),
            jax.ShapeDtypeStruct((16, G), jnp.float32),
        ],
    )(pa, pb)
    return jnp.concatenate([oa, obt.T, global_feats], axis=-1)


# speedup vs baseline: 5.6483x; 5.6483x over previous
import functools
import math

import jax
import jax.numpy as jnp
import numpy as np
from jax import lax
from jax.experimental import pallas as pl
from jax.experimental.pallas import tpu as pltpu
from jax.experimental.pallas import tpu_sc as plsc

MAXH = 3
G = 512
NCLS = 4
DUMP_A = NCLS * G
ACC_A_ROWS = 129 * 16
BROWS = 640
DUMP_ROW = 512
NC, NS = 2, 16
NW = NC * NS
CH = 128

_PI2 = np.float32(math.pi / 2.0)
W_HOP = tuple(
    float(np.float32(math.cos(np.float32(_PI2 * np.float32(h) / np.float32(3.0)))))
    for h in range(NCLS)
)


def _hclass(hp):
    return jnp.where((hp >= 1) & (hp <= MAXH), hp, 0)


def _atom_phase(w, feats, hop, seg, buf2, segb2, hopb2, idxb, acc, sems,
                n, S):
    lo = (((w * n) // NW) // 8) * 8
    hi = ((((w + 1) * n) // NW) // 8) * 8
    nfull = (hi - lo) // S
    max_rng = n // NW + 8
    nst = -(-max_rng // S)

    def abase(t):
        full = t < nfull
        base = pl.multiple_of(jnp.where(full, lo + t * S, hi - S), 8)
        done = jnp.where(full, base, lo + nfull * S)
        return base, done

    def issue(t, slot):
        @pl.when(t < nst)
        def _():
            base, _ = abase(t)
            pltpu.async_copy(feats.at[pl.ds(base, S)], buf2.at[slot],
                             sems.at[slot])
            pltpu.async_copy(seg.at[pl.ds(base, S)], segb2.at[slot],
                             sems.at[slot])
            pltpu.async_copy(hop.at[pl.ds(base, S)], hopb2.at[slot],
                             sems.at[slot])

    def compute(t, slot):
        @pl.when(t < nst)
        def _():
            pltpu.make_async_copy(
                feats.at[pl.ds(0, S)], buf2.at[slot], sems.at[slot]).wait()
            pltpu.make_async_copy(
                seg.at[pl.ds(0, S)], segb2.at[slot], sems.at[slot]).wait()
            pltpu.make_async_copy(
                hop.at[pl.ds(0, S)], hopb2.at[slot], sems.at[slot]).wait()
            base, done = abase(t)

            def idxk(k, c2):
                sg = segb2[slot, pl.ds(k * 16, 16)]
                hp = hopb2[slot, pl.ds(k * 16, 16)]
                row = _hclass(hp) * G + sg
                gidx = base + k * 16 + lax.iota(jnp.int32, 16)
                row = jnp.where(gidx < done, DUMP_A, row)
                idxb[k // 8, pl.ds((k % 8) * 16, 16)] = row
                return c2

            lax.fori_loop(0, S // 16, idxk, 0)
            for j in range(S // CH):
                pltpu.sync_copy(buf2.at[slot].at[pl.ds(j * CH, CH)],
                                acc.at[idxb.at[j]], add=True)

    issue(0, 0)

    def body(t2, carry):
        t = t2 * 2
        issue(t + 1, 1)
        compute(t, 0)
        issue(t + 2, 0)
        compute(t + 1, 1)
        return carry

    lax.fori_loop(0, (nst + 1) // 2, body, 0)


def _bond_phase(w, feats2, hop, seg, bbuf2, segb2, hopb2, tscr, wbuf,
                accl, sems, nblk, SB):
    lo = (w * nblk) // NW
    hi = ((w + 1) * nblk) // NW
    nfull = (hi - lo) // SB
    max_rng = nblk // NW + 1
    nst = -(-max_rng // SB)
    SROWS = SB * 8

    f16 = lax.iota(jnp.int32, 16)
    crow = (f16 // 8) * SROWS + (f16 % 8)
    i16 = lax.iota(jnp.int32, 16)
    m15 = i16 == 15
    half = nblk * 8
    zero16 = jnp.zeros((16,), jnp.float32)

    def flush(cur, accs):
        rowv = jnp.zeros((16,), jnp.int32) + cur
        for f in range(16):
            tot = plsc.cumsum(accs[f])
            plsc.addupdate_scatter(
                accl, [rowv, jnp.full((16,), f, jnp.int32)], tot, mask=m15)

    def bbase(t):
        full = t < nfull
        base = jnp.where(full, lo + t * SB, hi - SB)
        done = jnp.where(full, base, lo + nfull * SB) * 128
        return base, done

    def issue(t, slot):
        @pl.when(t < nst)
        def _():
            base, _ = bbase(t)
            for a in range(2):
                pltpu.async_copy(
                    feats2.at[pl.ds(a * half + base * 8, SROWS)],
                    bbuf2.at[slot].at[pl.ds(a * SROWS, SROWS)],
                    sems.at[slot])
            pltpu.async_copy(seg.at[pl.ds(base * 128, SB * 128)],
                             segb2.at[slot], sems.at[slot])
            pltpu.async_copy(hop.at[pl.ds(base * 128, SB * 128)],
                             hopb2.at[slot], sems.at[slot])

    def compute(t, slot, carry):
        def run(carry):
            for a in range(2):
                pltpu.make_async_copy(
                    feats2.at[pl.ds(0, SROWS)],
                    bbuf2.at[slot].at[pl.ds(a * SROWS, SROWS)],
                    sems.at[slot]).wait()
            pltpu.make_async_copy(
                seg.at[pl.ds(0, SB * 128)], segb2.at[slot],
                sems.at[slot]).wait()
            pltpu.make_async_copy(
                hop.at[pl.ds(0, SB * 128)], hopb2.at[slot],
                sems.at[slot]).wait()
            base, done = bbase(t)
            baseb = base * 128

            def tgtk(k, c2):
                j16 = k * 16
                sg = segb2[slot, pl.ds(j16, 16)]
                hp = hopb2[slot, pl.ds(j16, 16)]
                gidx = baseb + j16 + i16
                tscr[pl.ds(j16, 16)] = jnp.where(gidx < done, DUMP_ROW, sg)
                wv = jnp.where(
                    hp == 1, W_HOP[1],
                    jnp.where(hp == 2, W_HOP[2],
                              jnp.where(hp == 3, W_HOP[3], 1.0))).astype(
                                  jnp.float32)
                wbuf[pl.ds(j16, 16)] = wv
                return c2

            lax.fori_loop(0, SB * 8, tgtk, 0)

            def group(g, carry2):
                cur = carry2[0]
                accs = list(carry2[1:])
                j16 = g * 16
                tv16 = tscr[pl.ds(j16, 16)]
                t0 = tv16[0]
                t15 = tv16[15]
                w16 = wbuf[pl.ds(j16, 16)]
                rbase = (g // 8) * 8
                d0 = (g % 8) * 16

                def fast(cur, *accs):
                    same = t0 == cur

                    @pl.when(jnp.logical_not(same))
                    def _():
                        flush(cur, accs)

                    new = []
                    for f in range(16):
                        a, c = f // 8, f % 8
                        v = bbuf2[slot, a * SROWS + rbase + c,
                                  pl.ds(d0, 16)] * w16
                        new.append(jnp.where(same, accs[f] + v, v))
                    return (t0, *new)

                def slow(cur, *accs):
                    flush(cur, accs)
                    for u in range(16):
                        tgt = tv16[u]
                        wu = w16[u]
                        col = plsc.load_gather(
                            bbuf2.at[slot],
                            [crow + rbase,
                             jnp.full((16,), d0 + u, jnp.int32)])
                        plsc.addupdate(accl.at[tgt], col * wu)
                    return (jnp.int32(DUMP_ROW), *([zero16] * 16))

                return lax.cond(t0 == t15, fast, slow, cur, *accs)

            return lax.fori_loop(0, SB * 8, group, carry)

        return lax.cond(t < nst, run, lambda c: c, carry)

    issue(0, 0)

    def body(t2, carry):
        t = t2 * 2
        issue(t + 1, 1)
        carry = compute(t, 0, carry)
        issue(t + 2, 0)
        carry = compute(t + 1, 1, carry)
        return carry

    carry = lax.fori_loop(0, (nst + 1) // 2, body,
                          (jnp.int32(DUMP_ROW),) + (zero16,) * 16)
    flush(carry[0], list(carry[1:]))


@functools.lru_cache(maxsize=None)
def _make_sc(natom, nbond, da, db):
    sa = 128
    sbb = 8
    nblk = nbond // 128
    assert natom % 8 == 0 and nbond % 128 == 0
    assert natom // NW - 8 >= sa and nblk // NW >= sbb
    assert da == 128 and db == 16

    mesh = plsc.VectorSubcoreMesh(core_axis_name="c", subcore_axis_name="s")

    @functools.partial(
        pl.kernel,
        out_type=(
            jax.ShapeDtypeStruct((NC, NCLS * G, da), jnp.float32),
            jax.ShapeDtypeStruct((NC, BROWS, 16), jnp.float32),
        ),
        mesh=mesh,
        compiler_params=pltpu.CompilerParams(
            use_tc_tiling_on_sc=False, needs_layout_passes=False),
        scratch_types=[
            pltpu.VMEM((2, sa, da), jnp.float32),
            pltpu.VMEM((2, 2 * sbb * 8, 128), jnp.float32),
            pltpu.VMEM((2, sa), jnp.int32),
            pltpu.VMEM((2, sa), jnp.int32),
            pltpu.VMEM((2, sbb * 128), jnp.int32),
            pltpu.VMEM((2, sbb * 128), jnp.int32),
            pltpu.VMEM((sa // CH, CH), jnp.int32),
            pltpu.VMEM((BROWS, 16), jnp.float32),
            pltpu.VMEM((BROWS // 128, 128), jnp.int32),
            pltpu.VMEM((sbb * 128,), jnp.int32),
            pltpu.VMEM((sbb * 128,), jnp.float32),
            pltpu.SemaphoreType.DMA((2,)),
            pltpu.SemaphoreType.DMA((2,)),
            pltpu.VMEM_SHARED((ACC_A_ROWS, da), jnp.float32),
            pltpu.VMEM_SHARED((BROWS, 16), jnp.float32),
        ],
    )
    def sc(af, ah, asg, bf2, bh, bsg, za, zbt, pa, pb,
           abuf2, bbuf2, asegb2, ahopb2, bsegb2, bhopb2, idxb, accl, rix,
           tscr, wbuf, sema, semb, acc_a, acc_bt):
        c = lax.axis_index("c")
        s = lax.axis_index("s")
        w = c * NS + s
        pltpu.sync_copy(za, acc_a.at[pl.ds(s * 129, 129)])
        pltpu.sync_copy(zbt.at[pl.ds(s * (BROWS // 16), BROWS // 16)],
                        acc_bt.at[pl.ds(s * (BROWS // 16), BROWS // 16)])

        def zl(k, c2):
            accl[k, pl.ds(0, 16)] = jnp.zeros((16,), jnp.float32)
            return c2

        lax.fori_loop(0, BROWS, zl, 0)

        def zr(k, c2):
            rix[k // 8, pl.ds((k % 8) * 16, 16)] = k * 16 + lax.iota(
                jnp.int32, 16)
            return c2

        lax.fori_loop(0, BROWS // 16, zr, 0)
        plsc.subcore_barrier()

        _atom_phase(w, af, ah, asg, abuf2, asegb2, ahopb2, idxb, acc_a,
                    sema, natom, sa)
        _bond_phase(w, bf2, bh, bsg, bbuf2, bsegb2, bhopb2, tscr, wbuf,
                    accl, semb, nblk, sbb)
        for j in range(BROWS // 128):
            pltpu.sync_copy(accl.at[pl.ds(j * 128, 128)],
                            acc_bt.at[rix.at[j]], add=True)
        plsc.subcore_barrier()
        pltpu.sync_copy(acc_a.at[pl.ds(s * 128, 128)],
                        abuf2.at[0].at[pl.ds(0, 128)])
        pltpu.sync_copy(abuf2.at[0].at[pl.ds(0, 128)],
                        pa.at[c, pl.ds(s * 128, 128)])
        pltpu.sync_copy(acc_bt.at[pl.ds(s * (BROWS // 16), BROWS // 16)],
                        pb.at[c, pl.ds(s * (BROWS // 16), BROWS // 16)])

    return sc


def _combine_body(pa_ref, pb_ref, oa_ref, ob_ref):
    da = pa_ref.shape[-1]
    oa = jnp.zeros((G, da), jnp.float32)
    for c in range(NC):
        for h in range(NCLS):
            oa = oa + W_HOP[h] * pa_ref[c, h * G:(h + 1) * G, :]
    oa_ref[...] = oa
    ob_ref[...] = pb_ref[0, 0:G, :] + pb_ref[1, 0:G, :]


def kernel(atom_feats, bond_feats, global_feats, atom_hop_distance,
           bond_hop_distance, atom_segment_ids, bond_segment_ids):
    natom, da = atom_feats.shape
    nbond, db = bond_feats.shape
    nblk = nbond // 128
    bf2 = jnp.reshape(
        jnp.transpose(
            jnp.reshape(jnp.transpose(bond_feats), (2, 8, nblk, 128)),
            (0, 2, 1, 3)),
        (2 * nblk * 8, 128))
    za = jnp.zeros((129, da), jnp.float32)
    zbt = jnp.zeros((BROWS, 16), jnp.float32)
    sc = _make_sc(natom, nbond, da, db)
    pa, pb = sc(atom_feats, atom_hop_distance.astype(jnp.int32),
                atom_segment_ids.astype(jnp.int32), bf2,
                bond_hop_distance.astype(jnp.int32),
                bond_segment_ids.astype(jnp.int32), za, zbt)
    oa, ob = pl.pallas_call(
        _combine_body,
        out_shape=[
            jax.ShapeDtypeStruct((G, da), jnp.float32),
            jax.ShapeDtypeStruct((G, db), jnp.float32),
        ],
    )(pa, pb)
    return jnp.concatenate([oa, ob, global_feats], axis=-1)


# speedup vs baseline: 5.8609x; 1.0376x over previous
import functools
import math

import jax
import jax.numpy as jnp
import numpy as np
from jax import lax
from jax.experimental import pallas as pl
from jax.experimental.pallas import tpu as pltpu
from jax.experimental.pallas import tpu_sc as plsc

MAXH = 3
G = 512
NCLS = 4
DUMP_A = NCLS * G
ACC_A_ROWS = 129 * 16
BROWS = 640
DUMP_ROW = 512
NC, NS = 2, 16
NW = NC * NS
CH = 128

_PI2 = np.float32(math.pi / 2.0)
W_HOP = tuple(
    float(np.float32(math.cos(np.float32(_PI2 * np.float32(h) / np.float32(3.0)))))
    for h in range(NCLS)
)


def _hclass(hp):
    return jnp.where((hp >= 1) & (hp <= MAXH), hp, 0)


def _atom_phase(w, feats, hop, seg, buf2, segb2, hopb2, idxb, acc, sems,
                n, S):
    lo = (((w * n) // NW) // 8) * 8
    hi = ((((w + 1) * n) // NW) // 8) * 8
    nfull = (hi - lo) // S
    max_rng = n // NW + 8
    nst = -(-max_rng // S)

    def abase(t):
        full = t < nfull
        base = pl.multiple_of(jnp.where(full, lo + t * S, hi - S), 8)
        done = jnp.where(full, base, lo + nfull * S)
        return base, done

    def issue(t, slot):
        @pl.when(t < nst)
        def _():
            base, _ = abase(t)
            pltpu.async_copy(feats.at[pl.ds(base, S)], buf2.at[slot],
                             sems.at[slot])
            pltpu.async_copy(seg.at[pl.ds(base, S)], segb2.at[slot],
                             sems.at[slot])
            pltpu.async_copy(hop.at[pl.ds(base, S)], hopb2.at[slot],
                             sems.at[slot])

    def compute(t, slot):
        @pl.when(t < nst)
        def _():
            pltpu.make_async_copy(
                feats.at[pl.ds(0, S)], buf2.at[slot], sems.at[slot]).wait()
            pltpu.make_async_copy(
                seg.at[pl.ds(0, S)], segb2.at[slot], sems.at[slot]).wait()
            pltpu.make_async_copy(
                hop.at[pl.ds(0, S)], hopb2.at[slot], sems.at[slot]).wait()
            base, done = abase(t)

            def idxk(k, c2):
                sg = segb2[slot, pl.ds(k * 16, 16)]
                hp = hopb2[slot, pl.ds(k * 16, 16)]
                row = _hclass(hp) * G + sg
                gidx = base + k * 16 + lax.iota(jnp.int32, 16)
                row = jnp.where(gidx < done, DUMP_A, row)
                idxb[k // 8, pl.ds((k % 8) * 16, 16)] = row
                return c2

            lax.fori_loop(0, S // 16, idxk, 0)
            for j in range(S // CH):
                pltpu.sync_copy(buf2.at[slot].at[pl.ds(j * CH, CH)],
                                acc.at[idxb.at[j]], add=True)

    issue(0, 0)

    def body(t2, carry):
        t = t2 * 2
        issue(t + 1, 1)
        compute(t, 0)
        issue(t + 2, 0)
        compute(t + 1, 1)
        return carry

    lax.fori_loop(0, (nst + 1) // 2, body, 0)


def _bond_phase(w, feats2, hop, seg, bbuf2, segb2, hopb2,
                accl, sems, nblk, SB):
    lo = (w * nblk) // NW
    hi = ((w + 1) * nblk) // NW
    nfull = (hi - lo) // SB
    max_rng = nblk // NW + 1
    nst = -(-max_rng // SB)
    SROWS = SB * 8

    f16 = lax.iota(jnp.int32, 16)
    crow = (f16 // 8) * SROWS + (f16 % 8)
    i16 = lax.iota(jnp.int32, 16)
    m15 = i16 == 15
    half = nblk * 8
    zero16 = jnp.zeros((16,), jnp.float32)

    def flush(cur, accs):
        rowv = jnp.zeros((16,), jnp.int32) + cur
        for f in range(16):
            tot = plsc.cumsum(accs[f])
            plsc.addupdate_scatter(
                accl, [rowv, jnp.full((16,), f, jnp.int32)], tot, mask=m15)

    def bbase(t):
        full = t < nfull
        base = jnp.where(full, lo + t * SB, hi - SB)
        done = jnp.where(full, base, lo + nfull * SB) * 128
        return base, done

    def issue(t, slot):
        @pl.when(t < nst)
        def _():
            base, _ = bbase(t)
            for a in range(2):
                pltpu.async_copy(
                    feats2.at[pl.ds(a * half + base * 8, SROWS)],
                    bbuf2.at[slot].at[pl.ds(a * SROWS, SROWS)],
                    sems.at[slot])
            pltpu.async_copy(seg.at[pl.ds(base * 128, SB * 128)],
                             segb2.at[slot], sems.at[slot])
            pltpu.async_copy(hop.at[pl.ds(base * 128, SB * 128)],
                             hopb2.at[slot], sems.at[slot])

    def compute(t, slot, carry):
        def run(carry):
            for a in range(2):
                pltpu.make_async_copy(
                    feats2.at[pl.ds(0, SROWS)],
                    bbuf2.at[slot].at[pl.ds(a * SROWS, SROWS)],
                    sems.at[slot]).wait()
            pltpu.make_async_copy(
                seg.at[pl.ds(0, SB * 128)], segb2.at[slot],
                sems.at[slot]).wait()
            pltpu.make_async_copy(
                hop.at[pl.ds(0, SB * 128)], hopb2.at[slot],
                sems.at[slot]).wait()
            base, done = bbase(t)
            baseb = base * 128

            def group(g, carry2):
                cur = carry2[0]
                accs = list(carry2[1:])
                j16 = g * 16
                sg = segb2[slot, pl.ds(j16, 16)]
                hp = hopb2[slot, pl.ds(j16, 16)]
                gidx = baseb + j16 + i16
                tv16 = jnp.where(gidx < done, DUMP_ROW, sg)
                w16 = jnp.where(
                    hp == 1, W_HOP[1],
                    jnp.where(hp == 2, W_HOP[2],
                              jnp.where(hp == 3, W_HOP[3], 1.0))).astype(
                                  jnp.float32)
                t0 = tv16[0]
                t15 = tv16[15]
                rbase = (g // 8) * 8
                d0 = (g % 8) * 16

                def fast(cur, *accs):
                    same = t0 == cur

                    @pl.when(jnp.logical_not(same))
                    def _():
                        flush(cur, accs)

                    new = []
                    for f in range(16):
                        a, c = f // 8, f % 8
                        v = bbuf2[slot, a * SROWS + rbase + c,
                                  pl.ds(d0, 16)] * w16
                        new.append(jnp.where(same, accs[f] + v, v))
                    return (t0, *new)

                def slow(cur, *accs):
                    flush(cur, accs)
                    for u in range(16):
                        tgt = tv16[u]
                        wu = w16[u]
                        col = plsc.load_gather(
                            bbuf2.at[slot],
                            [crow + rbase,
                             jnp.full((16,), d0 + u, jnp.int32)])
                        plsc.addupdate(accl.at[tgt], col * wu)
                    return (jnp.int32(DUMP_ROW), *([zero16] * 16))

                return lax.cond(t0 == t15, fast, slow, cur, *accs)

            return lax.fori_loop(0, SB * 8, group, carry)

        return lax.cond(t < nst, run, lambda c: c, carry)

    issue(0, 0)

    def body(t2, carry):
        t = t2 * 2
        issue(t + 1, 1)
        carry = compute(t, 0, carry)
        issue(t + 2, 0)
        carry = compute(t + 1, 1, carry)
        return carry

    carry = lax.fori_loop(0, (nst + 1) // 2, body,
                          (jnp.int32(DUMP_ROW),) + (zero16,) * 16)
    flush(carry[0], list(carry[1:]))


@functools.lru_cache(maxsize=None)
def _make_sc(natom, nbond, da, db):
    sa = 128
    sbb = 8
    nblk = nbond // 128
    assert natom % 8 == 0 and nbond % 128 == 0
    assert natom // NW - 8 >= sa and nblk // NW >= sbb
    assert da == 128 and db == 16

    mesh = plsc.VectorSubcoreMesh(core_axis_name="c", subcore_axis_name="s")

    @functools.partial(
        pl.kernel,
        out_type=(
            jax.ShapeDtypeStruct((NC, NCLS * G, da), jnp.float32),
            jax.ShapeDtypeStruct((NC, BROWS, 16), jnp.float32),
        ),
        mesh=mesh,
        compiler_params=pltpu.CompilerParams(
            use_tc_tiling_on_sc=False, needs_layout_passes=False),
        scratch_types=[
            pltpu.VMEM((2, sa, da), jnp.float32),
            pltpu.VMEM((2, 2 * sbb * 8, 128), jnp.float32),
            pltpu.VMEM((2, sa), jnp.int32),
            pltpu.VMEM((2, sa), jnp.int32),
            pltpu.VMEM((2, sbb * 128), jnp.int32),
            pltpu.VMEM((2, sbb * 128), jnp.int32),
            pltpu.VMEM((sa // CH, CH), jnp.int32),
            pltpu.VMEM((BROWS, 16), jnp.float32),
            pltpu.VMEM((BROWS // 128, 128), jnp.int32),
            pltpu.SemaphoreType.DMA((2,)),
            pltpu.SemaphoreType.DMA((2,)),
            pltpu.VMEM_SHARED((ACC_A_ROWS, da), jnp.float32),
            pltpu.VMEM_SHARED((BROWS, 16), jnp.float32),
        ],
    )
    def sc(af, ah, asg, bf2, bh, bsg, za, zbt, pa, pb,
           abuf2, bbuf2, asegb2, ahopb2, bsegb2, bhopb2, idxb, accl, rix,
           sema, semb, acc_a, acc_bt):
        c = lax.axis_index("c")
        s = lax.axis_index("s")
        w = c * NS + s
        pltpu.sync_copy(za, acc_a.at[pl.ds(s * 129, 129)])
        pltpu.sync_copy(zbt.at[pl.ds(s * (BROWS // 16), BROWS // 16)],
                        acc_bt.at[pl.ds(s * (BROWS // 16), BROWS // 16)])

        def zl(k, c2):
            accl[k, pl.ds(0, 16)] = jnp.zeros((16,), jnp.float32)
            return c2

        lax.fori_loop(0, BROWS, zl, 0)

        def zr(k, c2):
            rix[k // 8, pl.ds((k % 8) * 16, 16)] = k * 16 + lax.iota(
                jnp.int32, 16)
            return c2

        lax.fori_loop(0, BROWS // 16, zr, 0)
        plsc.subcore_barrier()

        _atom_phase(w, af, ah, asg, abuf2, asegb2, ahopb2, idxb, acc_a,
                    sema, natom, sa)
        _bond_phase(w, bf2, bh, bsg, bbuf2, bsegb2, bhopb2,
                    accl, semb, nblk, sbb)
        for j in range(BROWS // 128):
            pltpu.sync_copy(accl.at[pl.ds(j * 128, 128)],
                            acc_bt.at[rix.at[j]], add=True)
        plsc.subcore_barrier()
        pltpu.sync_copy(acc_a.at[pl.ds(s * 128, 128)],
                        abuf2.at[0].at[pl.ds(0, 128)])
        pltpu.sync_copy(abuf2.at[0].at[pl.ds(0, 128)],
                        pa.at[c, pl.ds(s * 128, 128)])
        pltpu.sync_copy(acc_bt.at[pl.ds(s * (BROWS // 16), BROWS // 16)],
                        pb.at[c, pl.ds(s * (BROWS // 16), BROWS // 16)])

    return sc


def _combine_body(pa_ref, pb_ref, oa_ref, ob_ref):
    da = pa_ref.shape[-1]
    oa = jnp.zeros((G, da), jnp.float32)
    for c in range(NC):
        for h in range(NCLS):
            oa = oa + W_HOP[h] * pa_ref[c, h * G:(h + 1) * G, :]
    oa_ref[...] = oa
    ob_ref[...] = pb_ref[0, 0:G, :] + pb_ref[1, 0:G, :]


def kernel(atom_feats, bond_feats, global_feats, atom_hop_distance,
           bond_hop_distance, atom_segment_ids, bond_segment_ids):
    natom, da = atom_feats.shape
    nbond, db = bond_feats.shape
    nblk = nbond // 128
    bf2 = jnp.reshape(
        jnp.transpose(
            jnp.reshape(jnp.transpose(bond_feats), (2, 8, nblk, 128)),
            (0, 2, 1, 3)),
        (2 * nblk * 8, 128))
    za = jnp.zeros((129, da), jnp.float32)
    zbt = jnp.zeros((BROWS, 16), jnp.float32)
    sc = _make_sc(natom, nbond, da, db)
    pa, pb = sc(atom_feats, atom_hop_distance.astype(jnp.int32),
                atom_segment_ids.astype(jnp.int32), bf2,
                bond_hop_distance.astype(jnp.int32),
                bond_segment_ids.astype(jnp.int32), za, zbt)
    oa, ob = pl.pallas_call(
        _combine_body,
        out_shape=[
            jax.ShapeDtypeStruct((G, da), jnp.float32),
            jax.ShapeDtypeStruct((G, db), jnp.float32),
        ],
    )(pa, pb)
    return jnp.concatenate([oa, ob, global_feats], axis=-1)


# speedup vs baseline: 5.9716x; 1.0189x over previous
import functools
import math

import jax
import jax.numpy as jnp
import numpy as np
from jax import lax
from jax.experimental import pallas as pl
from jax.experimental.pallas import tpu as pltpu
from jax.experimental.pallas import tpu_sc as plsc

MAXH = 3
G = 512
NCLS = 4
DUMP_A = NCLS * G
ACC_A_ROWS = 129 * 16
BROWS = 640
DUMP_ROW = 512
NC, NS = 2, 16
NW = NC * NS
CH = 128

_PI2 = np.float32(math.pi / 2.0)
W_HOP = tuple(
    float(np.float32(math.cos(np.float32(_PI2 * np.float32(h) / np.float32(3.0)))))
    for h in range(NCLS)
)


def _hclass(hp):
    return jnp.where((hp >= 1) & (hp <= MAXH), hp, 0)


def _atom_phase(w, feats, hop, seg, buf2, segb2, hopb2, idxb, acc, sems,
                n, S):
    lo = (((w * n) // NW) // 8) * 8
    hi = ((((w + 1) * n) // NW) // 8) * 8
    nfull = (hi - lo) // S
    max_rng = n // NW + 8
    nst = -(-max_rng // S)

    def abase(t):
        full = t < nfull
        base = pl.multiple_of(jnp.where(full, lo + t * S, hi - S), 8)
        done = jnp.where(full, base, lo + nfull * S)
        return base, done

    def issue(t, slot):
        @pl.when(t < nst)
        def _():
            base, _ = abase(t)
            pltpu.async_copy(feats.at[pl.ds(base, S)], buf2.at[slot],
                             sems.at[slot])
            pltpu.async_copy(seg.at[pl.ds(base, S)], segb2.at[slot],
                             sems.at[slot])
            pltpu.async_copy(hop.at[pl.ds(base, S)], hopb2.at[slot],
                             sems.at[slot])

    def compute(t, slot):
        @pl.when(t < nst)
        def _():
            pltpu.make_async_copy(
                feats.at[pl.ds(0, S)], buf2.at[slot], sems.at[slot]).wait()
            pltpu.make_async_copy(
                seg.at[pl.ds(0, S)], segb2.at[slot], sems.at[slot]).wait()
            pltpu.make_async_copy(
                hop.at[pl.ds(0, S)], hopb2.at[slot], sems.at[slot]).wait()
            base, done = abase(t)

            def idxk(k, c2):
                sg = segb2[slot, pl.ds(k * 16, 16)]
                hp = hopb2[slot, pl.ds(k * 16, 16)]
                row = _hclass(hp) * G + sg
                gidx = base + k * 16 + lax.iota(jnp.int32, 16)
                row = jnp.where(gidx < done, DUMP_A, row)
                idxb[k // 8, pl.ds((k % 8) * 16, 16)] = row
                return c2

            lax.fori_loop(0, S // 16, idxk, 0, unroll=4)
            for j in range(S // CH):
                pltpu.sync_copy(buf2.at[slot].at[pl.ds(j * CH, CH)],
                                acc.at[idxb.at[j]], add=True)

    issue(0, 0)

    def body(t2, carry):
        t = t2 * 2
        issue(t + 1, 1)
        compute(t, 0)
        issue(t + 2, 0)
        compute(t + 1, 1)
        return carry

    lax.fori_loop(0, (nst + 1) // 2, body, 0)


def _bond_phase(w, feats2, hop, seg, bbuf2, segb2, hopb2,
                accl, sems, nblk, SB):
    lo = (w * nblk) // NW
    hi = ((w + 1) * nblk) // NW
    nfull = (hi - lo) // SB
    max_rng = nblk // NW + 1
    nst = -(-max_rng // SB)
    SROWS = SB * 8

    f16 = lax.iota(jnp.int32, 16)
    crow = (f16 // 8) * SROWS + (f16 % 8)
    i16 = lax.iota(jnp.int32, 16)
    m15 = i16 == 15
    half = nblk * 8
    zero16 = jnp.zeros((16,), jnp.float32)

    def flush(cur, accs):
        rowv = jnp.zeros((16,), jnp.int32) + cur
        for f in range(16):
            tot = plsc.cumsum(accs[f])
            plsc.addupdate_scatter(
                accl, [rowv, jnp.full((16,), f, jnp.int32)], tot, mask=m15)

    def bbase(t):
        full = t < nfull
        base = jnp.where(full, lo + t * SB, hi - SB)
        done = jnp.where(full, base, lo + nfull * SB) * 128
        return base, done

    def issue(t, slot):
        @pl.when(t < nst)
        def _():
            base, _ = bbase(t)
            for a in range(2):
                pltpu.async_copy(
                    feats2.at[pl.ds(a * half + base * 8, SROWS)],
                    bbuf2.at[slot].at[pl.ds(a * SROWS, SROWS)],
                    sems.at[slot])
            pltpu.async_copy(seg.at[pl.ds(base * 128, SB * 128)],
                             segb2.at[slot], sems.at[slot])
            pltpu.async_copy(hop.at[pl.ds(base * 128, SB * 128)],
                             hopb2.at[slot], sems.at[slot])

    def compute(t, slot, carry):
        def run(carry):
            for a in range(2):
                pltpu.make_async_copy(
                    feats2.at[pl.ds(0, SROWS)],
                    bbuf2.at[slot].at[pl.ds(a * SROWS, SROWS)],
                    sems.at[slot]).wait()
            pltpu.make_async_copy(
                seg.at[pl.ds(0, SB * 128)], segb2.at[slot],
                sems.at[slot]).wait()
            pltpu.make_async_copy(
                hop.at[pl.ds(0, SB * 128)], hopb2.at[slot],
                sems.at[slot]).wait()
            base, done = bbase(t)
            baseb = base * 128

            def group(g, carry2):
                cur = carry2[0]
                accs = list(carry2[1:])
                j16 = g * 16
                sg = segb2[slot, pl.ds(j16, 16)]
                hp = hopb2[slot, pl.ds(j16, 16)]
                gidx = baseb + j16 + i16
                tv16 = jnp.where(gidx < done, DUMP_ROW, sg)
                w16 = jnp.where(
                    hp == 1, W_HOP[1],
                    jnp.where(hp == 2, W_HOP[2],
                              jnp.where(hp == 3, W_HOP[3], 1.0))).astype(
                                  jnp.float32)
                t0 = tv16[0]
                t15 = tv16[15]
                rbase = (g // 8) * 8
                d0 = (g % 8) * 16

                def fast(cur, *accs):
                    same = t0 == cur

                    @pl.when(jnp.logical_not(same))
                    def _():
                        flush(cur, accs)

                    new = []
                    for f in range(16):
                        a, c = f // 8, f % 8
                        v = bbuf2[slot, a * SROWS + rbase + c,
                                  pl.ds(d0, 16)] * w16
                        new.append(jnp.where(same, accs[f] + v, v))
                    return (t0, *new)

                def slow(cur, *accs):
                    flush(cur, accs)
                    for u in range(16):
                        tgt = tv16[u]
                        wu = w16[u]
                        col = plsc.load_gather(
                            bbuf2.at[slot],
                            [crow + rbase,
                             jnp.full((16,), d0 + u, jnp.int32)])
                        plsc.addupdate(accl.at[tgt], col * wu)
                    return (jnp.int32(DUMP_ROW), *([zero16] * 16))

                return lax.cond(t0 == t15, fast, slow, cur, *accs)

            return lax.fori_loop(0, SB * 8, group, carry, unroll=2)

        return lax.cond(t < nst, run, lambda c: c, carry)

    issue(0, 0)

    def body(t2, carry):
        t = t2 * 2
        issue(t + 1, 1)
        carry = compute(t, 0, carry)
        issue(t + 2, 0)
        carry = compute(t + 1, 1, carry)
        return carry

    carry = lax.fori_loop(0, (nst + 1) // 2, body,
                          (jnp.int32(DUMP_ROW),) + (zero16,) * 16)
    flush(carry[0], list(carry[1:]))


@functools.lru_cache(maxsize=None)
def _make_sc(natom, nbond, da, db):
    sa = 128
    sbb = 8
    nblk = nbond // 128
    assert natom % 8 == 0 and nbond % 128 == 0
    assert natom // NW - 8 >= sa and nblk // NW >= sbb
    assert da == 128 and db == 16

    mesh = plsc.VectorSubcoreMesh(core_axis_name="c", subcore_axis_name="s")

    @functools.partial(
        pl.kernel,
        out_type=(
            jax.ShapeDtypeStruct((NC, NCLS * G, da), jnp.float32),
            jax.ShapeDtypeStruct((NC, BROWS, 16), jnp.float32),
        ),
        mesh=mesh,
        compiler_params=pltpu.CompilerParams(
            use_tc_tiling_on_sc=False, needs_layout_passes=False),
        scratch_types=[
            pltpu.VMEM((2, sa, da), jnp.float32),
            pltpu.VMEM((2, 2 * sbb * 8, 128), jnp.float32),
            pltpu.VMEM((2, sa), jnp.int32),
            pltpu.VMEM((2, sa), jnp.int32),
            pltpu.VMEM((2, sbb * 128), jnp.int32),
            pltpu.VMEM((2, sbb * 128), jnp.int32),
            pltpu.VMEM((sa // CH, CH), jnp.int32),
            pltpu.VMEM((BROWS, 16), jnp.float32),
            pltpu.VMEM((BROWS // 128, 128), jnp.int32),
            pltpu.SemaphoreType.DMA((2,)),
            pltpu.SemaphoreType.DMA((2,)),
            pltpu.VMEM_SHARED((ACC_A_ROWS, da), jnp.float32),
            pltpu.VMEM_SHARED((BROWS, 16), jnp.float32),
        ],
    )
    def sc(af, ah, asg, bf2, bh, bsg, za, zbt, pa, pb,
           abuf2, bbuf2, asegb2, ahopb2, bsegb2, bhopb2, idxb, accl, rix,
           sema, semb, acc_a, acc_bt):
        c = lax.axis_index("c")
        s = lax.axis_index("s")
        w = c * NS + s
        pltpu.sync_copy(za, acc_a.at[pl.ds(s * 129, 129)])
        pltpu.sync_copy(zbt.at[pl.ds(s * (BROWS // 16), BROWS // 16)],
                        acc_bt.at[pl.ds(s * (BROWS // 16), BROWS // 16)])

        def zl(k, c2):
            accl[k, pl.ds(0, 16)] = jnp.zeros((16,), jnp.float32)
            return c2

        lax.fori_loop(0, BROWS, zl, 0)

        def zr(k, c2):
            rix[k // 8, pl.ds((k % 8) * 16, 16)] = k * 16 + lax.iota(
                jnp.int32, 16)
            return c2

        lax.fori_loop(0, BROWS // 16, zr, 0)
        plsc.subcore_barrier()

        _atom_phase(w, af, ah, asg, abuf2, asegb2, ahopb2, idxb, acc_a,
                    sema, natom, sa)
        _bond_phase(w, bf2, bh, bsg, bbuf2, bsegb2, bhopb2,
                    accl, semb, nblk, sbb)
        for j in range(BROWS // 128):
            pltpu.sync_copy(accl.at[pl.ds(j * 128, 128)],
                            acc_bt.at[rix.at[j]], add=True)
        plsc.subcore_barrier()
        pltpu.sync_copy(acc_a.at[pl.ds(s * 128, 128)],
                        abuf2.at[0].at[pl.ds(0, 128)])
        pltpu.sync_copy(abuf2.at[0].at[pl.ds(0, 128)],
                        pa.at[c, pl.ds(s * 128, 128)])
        pltpu.sync_copy(acc_bt.at[pl.ds(s * (BROWS // 16), BROWS // 16)],
                        pb.at[c, pl.ds(s * (BROWS // 16), BROWS // 16)])

    return sc


def _combine_body(pa_ref, pb_ref, oa_ref, ob_ref):
    da = pa_ref.shape[-1]
    oa = jnp.zeros((G, da), jnp.float32)
    for c in range(NC):
        for h in range(NCLS):
            oa = oa + W_HOP[h] * pa_ref[c, h * G:(h + 1) * G, :]
    oa_ref[...] = oa
    ob_ref[...] = pb_ref[0, 0:G, :] + pb_ref[1, 0:G, :]


def kernel(atom_feats, bond_feats, global_feats, atom_hop_distance,
           bond_hop_distance, atom_segment_ids, bond_segment_ids):
    natom, da = atom_feats.shape
    nbond, db = bond_feats.shape
    nblk = nbond // 128
    bf2 = jnp.reshape(
        jnp.transpose(
            jnp.reshape(jnp.transpose(bond_feats), (2, 8, nblk, 128)),
            (0, 2, 1, 3)),
        (2 * nblk * 8, 128))
    za = jnp.zeros((129, da), jnp.float32)
    zbt = jnp.zeros((BROWS, 16), jnp.float32)
    sc = _make_sc(natom, nbond, da, db)
    pa, pb = sc(atom_feats, atom_hop_distance.astype(jnp.int32),
                atom_segment_ids.astype(jnp.int32), bf2,
                bond_hop_distance.astype(jnp.int32),
                bond_segment_ids.astype(jnp.int32), za, zbt)
    oa, ob = pl.pallas_call(
        _combine_body,
        out_shape=[
            jax.ShapeDtypeStruct((G, da), jnp.float32),
            jax.ShapeDtypeStruct((G, db), jnp.float32),
        ],
    )(pa, pb)
    return jnp.concatenate([oa, ob, global_feats], axis=-1)
